# Initial kernel scaffold; baseline (speedup 1.0000x reference)
import jax
import jax.numpy as jnp
from jax.experimental import pallas as pl


def kernel(x, w1, b1, w2, b2):
    raise NotImplementedError("write your pallas kernel here")


# speedup vs baseline: 1.1369x; 1.1369x over previous
from functools import partial

import jax
import jax.numpy as jnp
from jax.experimental import pallas as pl
from jax.experimental.pallas import tpu as pltpu


def _ca_fused_kernel(x_ref, w1t_ref, b1_ref, w2t_ref, b2_ref, o_ref, *,
                     inv_hw, hw):
    x = x_ref[...]
    if hw % 128 != 0:
        lane = jax.lax.broadcasted_iota(jnp.int32, x.shape, dimension=2)
        xz = jnp.where(lane < hw, x, 0.0)
    else:
        xz = x
    y = jnp.sum(xz, axis=-1, dtype=jnp.float32) * inv_hw

    t1 = jnp.dot(y, w1t_ref[...], preferred_element_type=jnp.float32)
    t1 = jnp.maximum(t1 + b1_ref[...], 0.0)
    t2 = jnp.dot(t1, w2t_ref[...], preferred_element_type=jnp.float32)
    scale = jax.nn.sigmoid(t2 + b2_ref[...]).astype(x_ref.dtype)

    o_ref[...] = (x_ref[...] * scale[:, :, None]).astype(o_ref.dtype)


def kernel(x, w1, b1, w2, b2):
    B, C, H, W = x.shape
    HW = H * W
    Cr = w1.shape[0]
    inv_hw = float(1.0 / HW)

    w1t = jnp.transpose(w1)
    w2t = jnp.transpose(w2)
    b1r = b1.reshape(1, Cr)
    b2r = b2.reshape(1, C)

    x_flat = x.reshape(B, C, HW)

    out_flat = pl.pallas_call(
        partial(_ca_fused_kernel, inv_hw=inv_hw, hw=HW),
        out_shape=jax.ShapeDtypeStruct((B, C, HW), x.dtype),
        grid=(B,),
        in_specs=[
            pl.BlockSpec((1, C, HW), lambda b: (b, 0, 0)),
            pl.BlockSpec((C, Cr), lambda b: (0, 0)),
            pl.BlockSpec((1, Cr), lambda b: (0, 0)),
            pl.BlockSpec((Cr, C), lambda b: (0, 0)),
            pl.BlockSpec((1, C), lambda b: (0, 0)),
        ],
        out_specs=pl.BlockSpec((1, C, HW), lambda b: (b, 0, 0)),
        compiler_params=pltpu.CompilerParams(
            dimension_semantics=("parallel",),
            vmem_limit_bytes=48 * 1024 * 1024,
        ),
    )(x_flat, w1t, b1r, w2t, b2r)

    return out_flat.reshape(B, C, H, W)
